# Initial kernel scaffold; baseline (speedup 1.0000x reference)
import jax
import jax.numpy as jnp
from jax.experimental import pallas as pl


def kernel(x, table):
    raise NotImplementedError("write your pallas kernel here")


# speedup vs baseline: 1.8795x; 1.8795x over previous
import functools

import jax
import jax.numpy as jnp
from jax import lax
from jax.experimental import pallas as pl
from jax.experimental.pallas import tpu as pltpu
from jax.experimental.pallas import tpu_sc as plsc

NC = 2
NS = 16
NW = NC * NS
CHUNK = 128
K = 5
SUPER = K * CHUNK


def _sc_gather(tot, d, dtype):
    per_w = tot // NW
    n_chunks = per_w // CHUNK
    n_super = n_chunks // K
    assert per_w * NW == tot and n_chunks * CHUNK == per_w
    assert n_super * K == n_chunks and n_super % 2 == 0

    mesh = plsc.VectorSubcoreMesh(
        core_axis_name="c", subcore_axis_name="s",
        num_cores=NC, num_subcores=NS)

    @functools.partial(
        pl.kernel,
        out_type=jax.ShapeDtypeStruct((NW, per_w, d), dtype),
        mesh=mesh,
        scratch_types=[
            pltpu.VMEM((n_chunks, CHUNK), jnp.int32),
            pltpu.VMEM((2, SUPER, d), dtype),
            pltpu.SemaphoreType.DMA,
            pltpu.SemaphoreType.DMA,
            pltpu.SemaphoreType.DMA,
            pltpu.SemaphoreType.DMA,
        ],
        compiler_params=pltpu.CompilerParams(use_tc_tiling_on_sc=False),
    )
    def run(tab_hbm, idx_hbm, out_hbm, idx_v, rows_v, g0, g1, s0, s1):
        wid = lax.axis_index("s") * NC + lax.axis_index("c")
        pltpu.sync_copy(idx_hbm.at[wid], idx_v)
        gsem = (g0, g1)
        ssem = (s0, s1)

        def fire_gathers(g, p):
            for b in range(K):
                pltpu.async_copy(
                    tab_hbm.at[idx_v.at[g * K + b]],
                    rows_v.at[p, pl.ds(b * CHUNK, CHUNK)],
                    gsem[p])

        def wait_gathers(g, p):
            for b in range(K):
                pltpu.make_async_copy(
                    tab_hbm.at[idx_v.at[g * K + b]],
                    rows_v.at[p, pl.ds(b * CHUNK, CHUNK)],
                    gsem[p]).wait()

        def fire_scatter(g, p):
            pltpu.async_copy(
                rows_v.at[p],
                out_hbm.at[wid, pl.ds(g * SUPER, SUPER)],
                ssem[p])

        def wait_scatter(p):
            pltpu.make_async_copy(
                rows_v.at[p],
                out_hbm.at[wid, pl.ds(0, SUPER)],
                ssem[p]).wait()

        fire_gathers(0, 0)
        wait_gathers(0, 0)
        fire_scatter(0, 0)
        fire_gathers(1, 1)

        @pl.loop(0, (n_super - 2) // 2)
        def _pair(t):
            for p, goff in ((1, 1), (0, 2)):
                g = 2 * t + goff
                wait_gathers(g, p)
                fire_scatter(g, p)
                wait_scatter(1 - p)
                fire_gathers(g + 1, 1 - p)

        wait_gathers(n_super - 1, 1)
        fire_scatter(n_super - 1, 1)
        wait_scatter(0)
        wait_scatter(1)

    return run


def kernel(x, table):
    b, h = x.shape
    n, d = table.shape
    tot = b * h
    idx = x.reshape(NW, tot // NW // CHUNK, CHUNK).astype(jnp.int32)
    out = _sc_gather(tot, d, table.dtype)(table, idx)
    return out.reshape(b, h, d)
